# Initial kernel scaffold; baseline (speedup 1.0000x reference)
import jax
import jax.numpy as jnp
from jax.experimental import pallas as pl


def kernel(inputs, embeddings):
    raise NotImplementedError("write your pallas kernel here")


# speedup vs baseline: 1.0781x; 1.0781x over previous
import functools

import jax
import jax.numpy as jnp
from jax import lax
from jax.experimental import pallas as pl
from jax.experimental.pallas import tpu as pltpu
from jax.experimental.pallas import tpu_sc as plsc

_D = 32
_B = 16384 * 50
_NC, _NS = 2, 16
_NW = _NC * _NS
_ROWS_PER_W = _B // _NW
_GRP = 128
_NGRP = _ROWS_PER_W // _GRP
_CG = 8
_NCHUNK = _NGRP // _CG


def _make_gather():
  mesh = plsc.VectorSubcoreMesh(core_axis_name="c", subcore_axis_name="s")

  @functools.partial(
      pl.kernel,
      out_type=jax.ShapeDtypeStruct((_NW * _NGRP, _GRP, _D), jnp.float32),
      mesh=mesh,
      scratch_types=[
          pltpu.VMEM((_NGRP, _GRP), jnp.int32),
          pltpu.VMEM((_CG, _GRP, _D), jnp.float32),
          pltpu.SemaphoreType.DMA,
      ],
      compiler_params=pltpu.CompilerParams(use_tc_tiling_on_sc=False),
  )
  def k(src_hbm, table_hbm, out_hbm, idx_v, rows_v, sem):
    wid = lax.axis_index("s") * _NC + lax.axis_index("c")
    pltpu.sync_copy(src_hbm.at[wid], idx_v)
    out_base = wid * _NGRP

    @pl.loop(0, _NCHUNK)
    def _chunk(c):
      descs = []
      for g in range(_CG):
        descs.append(
            pltpu.async_copy(
                table_hbm.at[idx_v.at[c * _CG + g]], rows_v.at[g], sem))
      for d in descs:
        d.wait()
      pltpu.sync_copy(rows_v, out_hbm.at[pl.ds(out_base + c * _CG, _CG)])

  return k


_gather = _make_gather()


def kernel(inputs, embeddings):
  src = jnp.reshape(inputs.astype(jnp.int32), (_NW, _NGRP, _GRP))
  out = _gather(src, embeddings)
  return jnp.reshape(out, (_B, _D))


# speedup vs baseline: 1.0954x; 1.0161x over previous
import functools

import jax
import jax.numpy as jnp
from jax import lax
from jax.experimental import pallas as pl
from jax.experimental.pallas import tpu as pltpu
from jax.experimental.pallas import tpu_sc as plsc

_D = 32
_B = 16384 * 50
_NC, _NS = 2, 16
_NW = _NC * _NS
_ROWS_PER_W = _B // _NW
_GRP = 128
_NGRP = _ROWS_PER_W // _GRP
_CG = 10
_NCHUNK = _NGRP // _CG


def _make_gather():
  mesh = plsc.VectorSubcoreMesh(core_axis_name="c", subcore_axis_name="s")

  @functools.partial(
      pl.kernel,
      out_type=jax.ShapeDtypeStruct((_NW * _NGRP, _GRP, _D), jnp.float32),
      mesh=mesh,
      scratch_types=[
          pltpu.VMEM((_NGRP, _GRP), jnp.int32),
          pltpu.VMEM((2, _CG, _GRP, _D), jnp.float32),
          pltpu.SemaphoreType.DMA,
          pltpu.SemaphoreType.DMA,
          pltpu.SemaphoreType.DMA,
          pltpu.SemaphoreType.DMA,
      ],
      compiler_params=pltpu.CompilerParams(use_tc_tiling_on_sc=False),
  )
  def k(src_hbm, table_hbm, out_hbm, idx_v, rows_v, gsem0, gsem1, osem0,
        osem1):
    wid = lax.axis_index("s") * _NC + lax.axis_index("c")
    pltpu.sync_copy(src_hbm.at[wid], idx_v)
    out_base = wid * _NGRP

    def fire_gathers(c, buf, sem):
      for g in range(_CG):
        pltpu.async_copy(
            table_hbm.at[idx_v.at[c * _CG + g]], rows_v.at[buf, g], sem)

    def drain(sem, buf):
      pltpu.make_async_copy(
          out_hbm.at[pl.ds(0, _CG)], rows_v.at[buf], sem).wait()

    def fire_out(c, buf, sem):
      pltpu.async_copy(
          rows_v.at[buf], out_hbm.at[pl.ds(out_base + c * _CG, _CG)], sem)

    fire_gathers(0, 0, gsem0)

    @pl.loop(0, _NCHUNK, step=2)
    def _chunks(c0):
      @pl.when(c0 >= 1)
      def _():
        drain(osem1, 1)
      fire_gathers(c0 + 1, 1, gsem1)
      drain(gsem0, 0)
      fire_out(c0, 0, osem0)
      @pl.when(c0 + 2 < _NCHUNK)
      def _():
        drain(osem0, 0)
        fire_gathers(c0 + 2, 0, gsem0)
      drain(gsem1, 1)
      fire_out(c0 + 1, 1, osem1)

    drain(osem0, 0)
    drain(osem1, 1)

  return k


_gather = _make_gather()


def kernel(inputs, embeddings):
  src = jnp.reshape(inputs.astype(jnp.int32), (_NW, _NGRP, _GRP))
  out = _gather(src, embeddings)
  return jnp.reshape(out, (_B, _D))
